# baseline (speedup 1.0000x reference)
import jax
import jax.numpy as jnp
from jax.experimental import pallas as pl
from jax.experimental.pallas import tpu as pltpu

_NUM_VARS = 64
_K = 32
_V = 256
_B = 512


def _circuit_body(inp_ref, leaf_ref, w1_ref, w2_ref, w3_ref, w4_ref,
                  w5_ref, w6_ref, wr_ref, out_ref,
                  b1, b2, b3, b4, b5, b6, sem):
    w_hbm = (w1_ref, w2_ref, w3_ref, w4_ref, w5_ref, w6_ref)
    w_buf = (b1, b2, b3, b4, b5, b6)
    copies = [pltpu.make_async_copy(w_hbm[l], w_buf[l], sem.at[l])
              for l in range(6)]
    for c in copies:
        c.start()

    iota_vb = jax.lax.broadcasted_iota(jnp.int32, (_V, _B), 0)
    ps = []
    for v in range(_NUM_VARS):
        leaf_v = leaf_ref[v]
        p_e = jnp.exp(leaf_v.astype(jnp.bfloat16))
        z_inv = 1.0 / jnp.sum(p_e, axis=1, keepdims=True,
                              dtype=jnp.float32)
        onehot = (iota_vb == inp_ref[v:v + 1, :]).astype(jnp.bfloat16)
        p_u = jnp.dot(p_e, onehot,
                      preferred_element_type=jnp.float32)
        ps.append((p_u * z_inv).astype(jnp.bfloat16))

    def mix(w_ref, r, e_l, e_r):
        prod = (e_l[:, None, :] * e_r[None, :, :]).reshape(_K * _K, _B)
        w_v = w_ref[r]
        w_e = jnp.exp(w_v.astype(jnp.bfloat16))
        z_w = 1.0 / jnp.sum(w_e, axis=1, keepdims=True,
                            dtype=jnp.float32)
        dot = jnp.dot(w_e, prod, preferred_element_type=jnp.float32)
        return dot * z_w

    copies[0].wait()
    ds = []
    for r in range(_K):
        ds.append(mix(b1, r, ps[2 * r], ps[2 * r + 1]))
    ms = [None] * _K

    for li, w_buf_l in enumerate((b2, b3, b4, b5, b6)):
        copies[li + 1].wait()
        nds, nms = [], []
        for r in range(w_buf_l.shape[0]):
            d_l, d_r = ds[2 * r], ds[2 * r + 1]
            u_l = jnp.max(d_l, axis=0, keepdims=True)
            u_r = jnp.max(d_r, axis=0, keepdims=True)
            e_l = (d_l * (1.0 / u_l)).astype(jnp.bfloat16)
            e_r = (d_r * (1.0 / u_r)).astype(jnp.bfloat16)
            m = jnp.log(u_l * u_r)
            if ms[2 * r] is not None:
                m = m + (ms[2 * r] + ms[2 * r + 1])
            nds.append(mix(w_buf_l, r, e_l, e_r))
            nms.append(m)
        ds, ms = nds, nms

    wr_col = wr_ref[...]
    e_wr = jnp.exp(wr_col)
    s = jnp.sum(ds[0] * e_wr, axis=0, keepdims=True)
    out_ref[...] = jnp.log(s) + ms[0] - jnp.log(jnp.sum(e_wr))


def kernel(inputs, leaf_logits, w1, w2, w3, w4, w5, w6, wr):
    vmem = pl.BlockSpec(memory_space=pltpu.MemorySpace.VMEM)
    hbm = pl.BlockSpec(memory_space=pltpu.MemorySpace.HBM)
    lls = pl.pallas_call(
        _circuit_body,
        out_shape=jax.ShapeDtypeStruct((1, _B), jnp.float32),
        in_specs=[vmem, vmem, hbm, hbm, hbm, hbm, hbm, hbm, vmem],
        scratch_shapes=[
            pltpu.VMEM((32, _K, _K * _K), jnp.float32),
            pltpu.VMEM((16, _K, _K * _K), jnp.float32),
            pltpu.VMEM((8, _K, _K * _K), jnp.float32),
            pltpu.VMEM((4, _K, _K * _K), jnp.float32),
            pltpu.VMEM((2, _K, _K * _K), jnp.float32),
            pltpu.VMEM((1, _K, _K * _K), jnp.float32),
            pltpu.SemaphoreType.DMA((6,)),
        ],
    )(inputs.T, leaf_logits, w1, w2, w3, w4, w5, w6, wr[:, None])
    return lls.reshape(_B, 1)


# speedup vs baseline: 1.0012x; 1.0012x over previous
import jax
import jax.numpy as jnp
from jax.experimental import pallas as pl
from jax.experimental.pallas import tpu as pltpu

_NUM_VARS = 64
_K = 32
_V = 256
_B = 512


def _circuit_body(inp_ref, leaf_ref, w1_ref, w2_ref, w3_ref, w4_ref,
                  w5_ref, w6_ref, wr_ref, out_ref,
                  b1, b2, b3, b4, b5, b6, sem):
    w_hbm = (w1_ref, w2_ref, w3_ref, w4_ref, w5_ref, w6_ref)
    w_buf = (b1, b2, b3, b4, b5, b6)
    copies = [pltpu.make_async_copy(w_hbm[l], w_buf[l], sem.at[l])
              for l in range(6)]
    for c in copies:
        c.start()

    iota_vb = jax.lax.broadcasted_iota(jnp.int32, (_V, _B), 0)
    ps = []
    for v in range(_NUM_VARS):
        leaf_v = leaf_ref[v]
        p_e = jnp.exp(leaf_v.astype(jnp.bfloat16))
        z_inv = 1.0 / jnp.sum(p_e, axis=1, keepdims=True,
                              dtype=jnp.float32)
        onehot = (iota_vb == inp_ref[v:v + 1, :]).astype(jnp.bfloat16)
        p_u = jnp.dot(p_e, onehot,
                      preferred_element_type=jnp.float32)
        ps.append((p_u * z_inv).astype(jnp.bfloat16))

    def mix(w_ref, r, prod):
        w_v = w_ref[r]
        w_e = jnp.exp(w_v.astype(jnp.bfloat16))
        lse_w = jnp.log(jnp.sum(w_e, axis=1, keepdims=True,
                                dtype=jnp.float32))
        dot = jnp.dot(w_e, prod, preferred_element_type=jnp.float32)
        return jnp.log(dot + 1e-37) - lse_w

    copies[0].wait()
    xs = []
    for r in range(_K):
        p_l = ps[2 * r]
        p_r = ps[2 * r + 1]
        prod = (p_l[:, None, :] * p_r[None, :, :]).reshape(_K * _K, _B)
        xs.append(mix(b1, r, prod))

    for li, w_buf_l in enumerate((b2, b3, b4, b5, b6)):
        copies[li + 1].wait()
        nxt = []
        for r in range(w_buf_l.shape[0]):
            lft = xs[2 * r]
            rgt = xs[2 * r + 1]
            m_l = jnp.max(lft, axis=0, keepdims=True)
            m_r = jnp.max(rgt, axis=0, keepdims=True)
            e_l = jnp.exp(lft - m_l).astype(jnp.bfloat16)
            e_r = jnp.exp(rgt - m_r).astype(jnp.bfloat16)
            prod = (e_l[:, None, :] * e_r[None, :, :]).reshape(_K * _K, _B)
            nxt.append(mix(w_buf_l, r, prod) + (m_l + m_r))
        xs = nxt

    wr_col = wr_ref[...]
    lse_w = jnp.log(jnp.sum(jnp.exp(wr_col)))
    t = xs[0] + (wr_col - lse_w)
    m_t = jnp.max(t, axis=0, keepdims=True)
    out_ref[...] = jnp.log(jnp.sum(jnp.exp(t - m_t), axis=0, keepdims=True)) + m_t


def kernel(inputs, leaf_logits, w1, w2, w3, w4, w5, w6, wr):
    vmem = pl.BlockSpec(memory_space=pltpu.MemorySpace.VMEM)
    hbm = pl.BlockSpec(memory_space=pltpu.MemorySpace.HBM)
    lls = pl.pallas_call(
        _circuit_body,
        out_shape=jax.ShapeDtypeStruct((1, _B), jnp.float32),
        in_specs=[vmem, vmem, hbm, hbm, hbm, hbm, hbm, hbm, vmem],
        scratch_shapes=[
            pltpu.VMEM((32, _K, _K * _K), jnp.float32),
            pltpu.VMEM((16, _K, _K * _K), jnp.float32),
            pltpu.VMEM((8, _K, _K * _K), jnp.float32),
            pltpu.VMEM((4, _K, _K * _K), jnp.float32),
            pltpu.VMEM((2, _K, _K * _K), jnp.float32),
            pltpu.VMEM((1, _K, _K * _K), jnp.float32),
            pltpu.SemaphoreType.DMA((6,)),
        ],
    )(inputs.T, leaf_logits, w1, w2, w3, w4, w5, w6, wr[:, None])
    return lls.reshape(_B, 1)
